# Initial kernel scaffold; baseline (speedup 1.0000x reference)
import functools

import jax
import jax.numpy as jnp
from jax import lax
from jax.experimental import pallas as pl
from jax.experimental.pallas import tpu as pltpu
from jax.experimental.pallas import tpu_sc as plsc

N = 10000
E = 320000
D = 128
H = 16
G = 64

NPAD = 16
NT = N + NPAD
NC = 2
NS = 16
NW = NC * NS
S = 128
KJ = 8
CH = KJ * S
EP = 327680
ROWS = EP // S
RPW = ROWS // NW
NCHUNK = RPW // KJ

_mesh = plsc.VectorSubcoreMesh(core_axis_name="c", subcore_axis_name="s")


def _sc_gather_deg_body(xl_hbm, xr_hbm, srcm, dstm, eam, zn,
                        gs_hbm, hd_hbm, degp, easump,
                        idxs, idxd, eab, ones, gsb, hdb, dacc, eacc,
                        sem1, sem2):
    c = lax.axis_index("c")
    s = lax.axis_index("s")
    wid = s * NC + c

    @pl.when(s == 0)
    def _():
        pltpu.sync_copy(zn, dacc)
        pltpu.sync_copy(zn, eacc)

    for i in range(S // 16):
        ones[pl.ds(i * 16, 16)] = jnp.full((16,), 1.0, jnp.float32)
    plsc.subcore_barrier()

    row0 = wid * RPW

    @pl.loop(0, NCHUNK)
    def _(chk):
        r = row0 + chk * KJ
        e0 = r * S
        pltpu.sync_copy(srcm.at[pl.ds(r, KJ)], idxs)
        pltpu.sync_copy(dstm.at[pl.ds(r, KJ)], idxd)
        pltpu.sync_copy(eam.at[pl.ds(r, KJ)], eab)
        cps = []
        for j in range(KJ):
            cps.append(pltpu.async_copy(
                xl_hbm.at[idxs.at[j]], gsb.at[pl.ds(j * S, S)], sem1))
            cps.append(pltpu.async_copy(
                xr_hbm.at[idxd.at[j]], hdb.at[pl.ds(j * S, S)], sem2))
        for j in range(KJ):
            pltpu.sync_copy(ones, dacc.at[idxd.at[j]], add=True)
            pltpu.sync_copy(eab.at[j], eacc.at[idxd.at[j]], add=True)
        for cp in cps:
            cp.wait()
        pltpu.sync_copy(gsb, gs_hbm.at[pl.ds(e0, CH)])
        pltpu.sync_copy(hdb, hd_hbm.at[pl.ds(e0, CH)])

    plsc.subcore_barrier()

    @pl.when(s == 0)
    def _():
        pltpu.sync_copy(dacc, degp.at[c])
        pltpu.sync_copy(eacc, easump.at[c])


def _sc_gather_deg(xl_t, xr_t, srcm, dstm, eam, zn):
    f = functools.partial(
        pl.kernel,
        out_type=[
            jax.ShapeDtypeStruct((EP, H), jnp.float32),
            jax.ShapeDtypeStruct((EP, H), jnp.float32),
            jax.ShapeDtypeStruct((NC, NT), jnp.float32),
            jax.ShapeDtypeStruct((NC, NT), jnp.float32),
        ],
        mesh=_mesh,
        scratch_types=[
            pltpu.VMEM((KJ, S), jnp.int32),
            pltpu.VMEM((KJ, S), jnp.int32),
            pltpu.VMEM((KJ, S), jnp.float32),
            pltpu.VMEM((S,), jnp.float32),
            pltpu.VMEM((CH, H), jnp.float32),
            pltpu.VMEM((CH, H), jnp.float32),
            pltpu.VMEM_SHARED((NT,), jnp.float32),
            pltpu.VMEM_SHARED((NT,), jnp.float32),
            pltpu.SemaphoreType.DMA,
            pltpu.SemaphoreType.DMA,
        ],
    )
    return f(_sc_gather_deg_body)(xl_t, xr_t, srcm, dstm, eam, zn)


def _sc_gather_body(xl_hbm, xr_hbm, srcm, dstm, gs_hbm, hd_hbm,
                    idxs, idxd, gsb, hdb, sem1, sem2):
    c = lax.axis_index("c")
    s = lax.axis_index("s")
    wid = s * NC + c
    row0 = wid * RPW

    @pl.loop(0, NCHUNK)
    def _(chk):
        r = row0 + chk * KJ
        e0 = r * S
        pltpu.sync_copy(srcm.at[pl.ds(r, KJ)], idxs)
        pltpu.sync_copy(dstm.at[pl.ds(r, KJ)], idxd)
        cps = []
        for j in range(KJ):
            cps.append(pltpu.async_copy(
                xl_hbm.at[idxs.at[j]], gsb.at[pl.ds(j * S, S)], sem1))
            cps.append(pltpu.async_copy(
                xr_hbm.at[idxd.at[j]], hdb.at[pl.ds(j * S, S)], sem2))
        for cp in cps:
            cp.wait()
        pltpu.sync_copy(gsb, gs_hbm.at[pl.ds(e0, CH)])
        pltpu.sync_copy(hdb, hd_hbm.at[pl.ds(e0, CH)])


def _sc_gather(xl_t, xr_t, srcm, dstm):
    f = functools.partial(
        pl.kernel,
        out_type=[
            jax.ShapeDtypeStruct((EP, H), jnp.float32),
            jax.ShapeDtypeStruct((EP, H), jnp.float32),
        ],
        mesh=_mesh,
        scratch_types=[
            pltpu.VMEM((KJ, S), jnp.int32),
            pltpu.VMEM((KJ, S), jnp.int32),
            pltpu.VMEM((CH, H), jnp.float32),
            pltpu.VMEM((CH, H), jnp.float32),
            pltpu.SemaphoreType.DMA,
            pltpu.SemaphoreType.DMA,
        ],
    )
    return f(_sc_gather_body)(xl_t, xr_t, srcm, dstm)


def _sc_scatter_body(wm, am, dstm, zn16, zn, msgp, denp,
                     idxd, ab, wb, macc, dnacc, sem1):
    c = lax.axis_index("c")
    s = lax.axis_index("s")
    wid = s * NC + c

    @pl.when(s == 0)
    def _():
        pltpu.sync_copy(zn16, macc)
        pltpu.sync_copy(zn, dnacc)
    plsc.subcore_barrier()

    row0 = wid * RPW

    @pl.loop(0, NCHUNK)
    def _(chk):
        r = row0 + chk * KJ
        e0 = r * S
        pltpu.sync_copy(dstm.at[pl.ds(r, KJ)], idxd)
        pltpu.sync_copy(am.at[pl.ds(r, KJ)], ab)
        pltpu.sync_copy(wm.at[pl.ds(e0, CH)], wb)
        for j in range(KJ):
            pltpu.sync_copy(wb.at[pl.ds(j * S, S)], macc.at[idxd.at[j]],
                            add=True)
            pltpu.sync_copy(ab.at[j], dnacc.at[idxd.at[j]], add=True)

    plsc.subcore_barrier()

    @pl.when(s == 0)
    def _():
        pltpu.sync_copy(macc, msgp.at[c])
        pltpu.sync_copy(dnacc, denp.at[c])


def _sc_scatter(wm, am, dstm, zn16, zn):
    f = functools.partial(
        pl.kernel,
        out_type=[
            jax.ShapeDtypeStruct((NC, NT, H), jnp.float32),
            jax.ShapeDtypeStruct((NC, NT), jnp.float32),
        ],
        mesh=_mesh,
        scratch_types=[
            pltpu.VMEM((KJ, S), jnp.int32),
            pltpu.VMEM((KJ, S), jnp.float32),
            pltpu.VMEM((CH, H), jnp.float32),
            pltpu.VMEM_SHARED((NT, H), jnp.float32),
            pltpu.VMEM_SHARED((NT,), jnp.float32),
            pltpu.SemaphoreType.DMA,
        ],
    )
    return f(_sc_scatter_body)(wm, am, dstm, zn16, zn)


def _node_body(x_ref, wl_ref, bl_ref, wr_ref, br_ref, xl_ref, xr_ref):
    x = x_ref[...]
    xl_ref[...] = jnp.dot(x, wl_ref[...],
                          preferred_element_type=jnp.float32) + bl_ref[...]
    xr_ref[...] = jnp.dot(x, wr_ref[...],
                          preferred_element_type=jnp.float32) + br_ref[...]


def _tc_node(x, wl, bl, wr, br):
    return pl.pallas_call(
        _node_body,
        out_shape=[jax.ShapeDtypeStruct((N, H), jnp.float32),
                   jax.ShapeDtypeStruct((N, H), jnp.float32)],
    )(x, wl, bl, wr, br)


_EB = 2048


def _edge_body(gs_ref, hd_ref, ea_ref, we_ref, att_ref, w_ref, a_ref):
    gs = gs_ref[...]
    m = gs + hd_ref[...] + ea_ref[...] * we_ref[...]
    m = jnp.where(m >= 0, m, 0.2 * m)
    a = jnp.exp(jnp.sum(m * att_ref[...], axis=1, keepdims=True))
    w_ref[...] = gs * a
    a_ref[...] = a


def _tc_edge(gs, hd, ea, we, att):
    grid = (EP // _EB,)
    return pl.pallas_call(
        _edge_body,
        grid=grid,
        in_specs=[
            pl.BlockSpec((_EB, H), lambda i: (i, 0)),
            pl.BlockSpec((_EB, H), lambda i: (i, 0)),
            pl.BlockSpec((_EB, 1), lambda i: (i, 0)),
            pl.BlockSpec((1, H), lambda i: (0, 0)),
            pl.BlockSpec((1, H), lambda i: (0, 0)),
        ],
        out_specs=[
            pl.BlockSpec((_EB, H), lambda i: (i, 0)),
            pl.BlockSpec((_EB, 1), lambda i: (i, 0)),
        ],
        out_shape=[jax.ShapeDtypeStruct((EP, H), jnp.float32),
                   jax.ShapeDtypeStruct((EP, 1), jnp.float32)],
    )(gs, hd, ea, we, att)


_NB = 2000


def _comb1_body(m0, m1, d0, d1, dg0, dg1, es0, es1, xl, xr,
                we, att, b1, wl2, bl2, wr2, br2,
                xl2o, xr2o, lao):
    deg = dg0[...] + dg1[...]
    la = (es0[...] + es1[...]) / jnp.maximum(deg, 1.0)
    xlv = xl[...]
    ms = xlv + xr[...] + la * we[...]
    ms = jnp.where(ms >= 0, ms, 0.2 * ms)
    aself = jnp.exp(jnp.sum(ms * att[...], axis=1, keepdims=True))
    msg = m0[...] + m1[...] + aself * xlv
    den = d0[...] + d1[...] + aself
    h = msg / (den + 1e-16) + b1[...]
    h = jnp.where(h > 0, h, jnp.expm1(h))
    xl2o[...] = jnp.dot(h, wl2[...],
                        preferred_element_type=jnp.float32) + bl2[...]
    xr2o[...] = jnp.dot(h, wr2[...],
                        preferred_element_type=jnp.float32) + br2[...]
    lao[...] = la


def _tc_comb1(m0, m1, d0, d1, dg0, dg1, es0, es1, xl, xr,
              we, att, b1, wl2, bl2, wr2, br2):
    grid = (N // _NB,)
    nb = lambda w: pl.BlockSpec((_NB, w), lambda i: (i, 0))
    full = lambda a, b: pl.BlockSpec((a, b), lambda i: (0, 0))
    return pl.pallas_call(
        _comb1_body,
        grid=grid,
        in_specs=[
            nb(H), nb(H), nb(1), nb(1), nb(1), nb(1), nb(1), nb(1),
            nb(H), nb(H),
            full(1, H), full(1, H), full(1, H),
            full(H, H), full(1, H), full(H, H), full(1, H),
        ],
        out_specs=[nb(H), nb(H), nb(1)],
        out_shape=[jax.ShapeDtypeStruct((N, H), jnp.float32),
                   jax.ShapeDtypeStruct((N, H), jnp.float32),
                   jax.ShapeDtypeStruct((N, 1), jnp.float32)],
    )(m0, m1, d0, d1, dg0, dg1, es0, es1, xl, xr,
      we, att, b1, wl2, bl2, wr2, br2)


def _final_body(m0, m1, d0, d1, la, xl, xr, bt, we, att, b2, w3, b3,
                out_ref):
    xlv = xl[...]
    ms = xlv + xr[...] + la[...] * we[...]
    ms = jnp.where(ms >= 0, ms, 0.2 * ms)
    aself = jnp.exp(jnp.sum(ms * att[...], axis=1, keepdims=True))
    msg = m0[...] + m1[...] + aself * xlv
    den = d0[...] + d1[...] + aself
    h = msg / (den + 1e-16) + b2[...]
    h = jnp.where(h > 0, h, jnp.expm1(h))
    onehot = (bt[...] == lax.broadcasted_iota(jnp.int32, (G, N), 0)
              ).astype(jnp.float32)
    cnt = jnp.sum(onehot, axis=1, keepdims=True)
    pooled = jax.lax.dot_general(onehot, h, (((1,), (0,)), ((), ())),
                                 preferred_element_type=jnp.float32)
    pooled = pooled / jnp.maximum(cnt, 1.0)
    out_ref[...] = jnp.dot(pooled, w3[...],
                           preferred_element_type=jnp.float32) + b3[...]


def _tc_final(m0, m1, d0, d1, la, xl, xr, bt, we, att, b2, w3, b3):
    return pl.pallas_call(
        _final_body,
        out_shape=jax.ShapeDtypeStruct((G, 1), jnp.float32),
    )(m0, m1, d0, d1, la, xl, xr, bt, we, att, b2, w3, b3)


def kernel(x, edge_index, edge_attr, batch,
           Wl1, bl1, Wr1, br1, We1, att1, b1,
           Wl2, bl2, Wr2, br2, We2, att2, b2,
           W3, b3):
    f32 = jnp.float32
    src = edge_index[0]
    dst = edge_index[1]

    npad_e = EP - E
    padtgt = (N + (jnp.arange(npad_e, dtype=jnp.int32) % NPAD))
    src_p = jnp.concatenate([src, padtgt])
    dst_p = jnp.concatenate([dst, padtgt])
    ea_p = jnp.concatenate([edge_attr[:, 0], jnp.zeros((npad_e,), f32)])
    srcm = src_p.reshape(ROWS, S)
    dstm = dst_p.reshape(ROWS, S)
    eam = ea_p.reshape(ROWS, S)
    ea_col = ea_p.reshape(EP, 1)

    zn = jnp.zeros((NT,), f32)
    zn16 = jnp.zeros((NT, H), f32)
    zrows = jnp.zeros((NPAD, H), f32)

    H2 = Wl2.shape[1]
    pw = ((0, 0), (0, H - H2))
    Wl2p = jnp.pad(Wl2, pw)
    Wr2p = jnp.pad(Wr2, pw)
    bl2p = jnp.pad(bl2, (0, H - H2)).reshape(1, H)
    br2p = jnp.pad(br2, (0, H - H2)).reshape(1, H)
    We2p = jnp.pad(We2, pw).reshape(1, H)
    att2p = jnp.pad(att2, (0, H - H2)).reshape(1, H)
    b2p = jnp.pad(b2, (0, H - H2)).reshape(1, H)
    W3p = jnp.pad(W3, ((0, H - H2), (0, 0)))

    We1r = We1.reshape(1, H)
    att1r = att1.reshape(1, H)
    b1r = b1.reshape(1, H)
    bl1r = bl1.reshape(1, H)
    br1r = br1.reshape(1, H)

    xl1, xr1 = _tc_node(x, Wl1, bl1r, Wr1, br1r)
    xl1t = jnp.concatenate([xl1, zrows])
    xr1t = jnp.concatenate([xr1, zrows])
    gs1, hd1, degp, easump = _sc_gather_deg(xl1t, xr1t, srcm, dstm, eam, zn)
    w1, a1 = _tc_edge(gs1, hd1, ea_col, We1r, att1r)
    msgp1, denp1 = _sc_scatter(w1, a1.reshape(ROWS, S), dstm, zn16, zn)

    dg0 = degp[0, :N].reshape(N, 1)
    dg1 = degp[1, :N].reshape(N, 1)
    es0 = easump[0, :N].reshape(N, 1)
    es1 = easump[1, :N].reshape(N, 1)
    m0 = msgp1[0, :N]
    m1 = msgp1[1, :N]
    d0 = denp1[0, :N].reshape(N, 1)
    d1 = denp1[1, :N].reshape(N, 1)

    xl2, xr2, la = _tc_comb1(m0, m1, d0, d1, dg0, dg1, es0, es1, xl1, xr1,
                             We1r, att1r, b1r, Wl2p, bl2p, Wr2p, br2p)

    xl2t = jnp.concatenate([xl2, zrows])
    xr2t = jnp.concatenate([xr2, zrows])
    gs2, hd2 = _sc_gather(xl2t, xr2t, srcm, dstm)
    w2, a2 = _tc_edge(gs2, hd2, ea_col, We2p, att2p)
    msgp2, denp2 = _sc_scatter(w2, a2.reshape(ROWS, S), dstm, zn16, zn)

    m0b = msgp2[0, :N]
    m1b = msgp2[1, :N]
    d0b = denp2[0, :N].reshape(N, 1)
    d1b = denp2[1, :N].reshape(N, 1)

    bt = batch.reshape(1, N)
    out = _tc_final(m0b, m1b, d0b, d1b, la, xl2, xr2, bt,
                    We2p, att2p, b2p, W3p, b3.reshape(1, 1))
    return out


# speedup vs baseline: 27.0967x; 27.0967x over previous
import functools

import jax
import jax.numpy as jnp
from jax import lax
from jax.experimental import pallas as pl
from jax.experimental.pallas import tpu as pltpu
from jax.experimental.pallas import tpu_sc as plsc

N = 10000
E = 320000
D = 128
H = 16
G = 64

NPAD = 16
NT = N + NPAD
NC = 2
NS = 16
NW = NC * NS
S = 128
KJ = 8
CH = KJ * S
EP = 327680
ROWS = EP // S
RPW = ROWS // NW
NCHUNK = RPW // KJ

FR = EP * H // 128

_mesh = plsc.VectorSubcoreMesh(core_axis_name="c", subcore_axis_name="s")
_sc_params = pltpu.CompilerParams(use_tc_tiling_on_sc=False)


def _sc_gather_deg_body(xl_hbm, xr_hbm, srcm, dstm, eam, zn,
                        gs_hbm, hd_hbm, degp, easump,
                        idxs, idxd, eab, ones, gsb, hdb, dacc, eacc,
                        sem1, sem2):
    c = lax.axis_index("c")
    s = lax.axis_index("s")
    wid = s * NC + c

    @pl.when(s == 0)
    def _():
        pltpu.sync_copy(zn, dacc)
        pltpu.sync_copy(zn, eacc)

    for i in range(S // 16):
        ones[pl.ds(i * 16, 16)] = jnp.full((16,), 1.0, jnp.float32)
    plsc.subcore_barrier()

    row0 = wid * RPW

    @pl.loop(0, NCHUNK)
    def _(chk):
        r = row0 + chk * KJ
        e0 = r * S
        pltpu.sync_copy(srcm.at[pl.ds(r, KJ)], idxs)
        pltpu.sync_copy(dstm.at[pl.ds(r, KJ)], idxd)
        pltpu.sync_copy(eam.at[pl.ds(r, KJ)], eab)
        cps = []
        for j in range(KJ):
            cps.append(pltpu.async_copy(
                xl_hbm.at[idxs.at[j]], gsb.at[pl.ds(j * S, S)], sem1))
            cps.append(pltpu.async_copy(
                xr_hbm.at[idxd.at[j]], hdb.at[pl.ds(j * S, S)], sem2))
        for j in range(KJ):
            pltpu.sync_copy(ones, dacc.at[idxd.at[j]], add=True)
            pltpu.sync_copy(eab.at[j], eacc.at[idxd.at[j]], add=True)
        for cp in cps:
            cp.wait()
        pltpu.sync_copy(gsb, gs_hbm.at[pl.ds(e0, CH)])
        pltpu.sync_copy(hdb, hd_hbm.at[pl.ds(e0, CH)])

    plsc.subcore_barrier()

    @pl.when(s == 0)
    def _():
        pltpu.sync_copy(dacc, degp.at[c])
        pltpu.sync_copy(eacc, easump.at[c])


def _sc_gather_deg(xl_t, xr_t, srcm, dstm, eam, zn):
    f = functools.partial(
        pl.kernel,
        out_type=[
            jax.ShapeDtypeStruct((EP, H), jnp.float32),
            jax.ShapeDtypeStruct((EP, H), jnp.float32),
            jax.ShapeDtypeStruct((NC, NT), jnp.float32),
            jax.ShapeDtypeStruct((NC, NT), jnp.float32),
        ],
        mesh=_mesh,
        compiler_params=_sc_params,
        scratch_types=[
            pltpu.VMEM((KJ, S), jnp.int32),
            pltpu.VMEM((KJ, S), jnp.int32),
            pltpu.VMEM((KJ, S), jnp.float32),
            pltpu.VMEM((S,), jnp.float32),
            pltpu.VMEM((CH, H), jnp.float32),
            pltpu.VMEM((CH, H), jnp.float32),
            pltpu.VMEM_SHARED((NT,), jnp.float32),
            pltpu.VMEM_SHARED((NT,), jnp.float32),
            pltpu.SemaphoreType.DMA,
            pltpu.SemaphoreType.DMA,
        ],
    )
    return f(_sc_gather_deg_body)(xl_t, xr_t, srcm, dstm, eam, zn)


def _sc_gather_body(xl_hbm, xr_hbm, srcm, dstm, gs_hbm, hd_hbm,
                    idxs, idxd, gsb, hdb, sem1, sem2):
    c = lax.axis_index("c")
    s = lax.axis_index("s")
    wid = s * NC + c
    row0 = wid * RPW

    @pl.loop(0, NCHUNK)
    def _(chk):
        r = row0 + chk * KJ
        e0 = r * S
        pltpu.sync_copy(srcm.at[pl.ds(r, KJ)], idxs)
        pltpu.sync_copy(dstm.at[pl.ds(r, KJ)], idxd)
        cps = []
        for j in range(KJ):
            cps.append(pltpu.async_copy(
                xl_hbm.at[idxs.at[j]], gsb.at[pl.ds(j * S, S)], sem1))
            cps.append(pltpu.async_copy(
                xr_hbm.at[idxd.at[j]], hdb.at[pl.ds(j * S, S)], sem2))
        for cp in cps:
            cp.wait()
        pltpu.sync_copy(gsb, gs_hbm.at[pl.ds(e0, CH)])
        pltpu.sync_copy(hdb, hd_hbm.at[pl.ds(e0, CH)])


def _sc_gather(xl_t, xr_t, srcm, dstm):
    f = functools.partial(
        pl.kernel,
        out_type=[
            jax.ShapeDtypeStruct((EP, H), jnp.float32),
            jax.ShapeDtypeStruct((EP, H), jnp.float32),
        ],
        mesh=_mesh,
        compiler_params=_sc_params,
        scratch_types=[
            pltpu.VMEM((KJ, S), jnp.int32),
            pltpu.VMEM((KJ, S), jnp.int32),
            pltpu.VMEM((CH, H), jnp.float32),
            pltpu.VMEM((CH, H), jnp.float32),
            pltpu.SemaphoreType.DMA,
            pltpu.SemaphoreType.DMA,
        ],
    )
    return f(_sc_gather_body)(xl_t, xr_t, srcm, dstm)


def _sc_scatter_body(wm, am, dstm, zn16, msgp, denp,
                     idxd, ab, wb, macc, dnacc, sem1):
    c = lax.axis_index("c")
    s = lax.axis_index("s")
    wid = s * NC + c

    @pl.when(s == 0)
    def _():
        pltpu.sync_copy(zn16, macc)
        pltpu.sync_copy(zn16, dnacc)
    plsc.subcore_barrier()

    row0 = wid * RPW

    @pl.loop(0, NCHUNK)
    def _(chk):
        r = row0 + chk * KJ
        e0 = r * S
        pltpu.sync_copy(dstm.at[pl.ds(r, KJ)], idxd)
        pltpu.sync_copy(am.at[pl.ds(e0, CH)], ab)
        pltpu.sync_copy(wm.at[pl.ds(e0, CH)], wb)
        for j in range(KJ):
            pltpu.sync_copy(wb.at[pl.ds(j * S, S)], macc.at[idxd.at[j]],
                            add=True)
            pltpu.sync_copy(ab.at[pl.ds(j * S, S)], dnacc.at[idxd.at[j]],
                            add=True)

    plsc.subcore_barrier()

    @pl.when(s == 0)
    def _():
        pltpu.sync_copy(macc, msgp.at[c])
        pltpu.sync_copy(dnacc, denp.at[c])


def _sc_scatter(wm, am, dstm, zn16):
    f = functools.partial(
        pl.kernel,
        out_type=[
            jax.ShapeDtypeStruct((NC, NT, H), jnp.float32),
            jax.ShapeDtypeStruct((NC, NT, H), jnp.float32),
        ],
        mesh=_mesh,
        compiler_params=_sc_params,
        scratch_types=[
            pltpu.VMEM((KJ, S), jnp.int32),
            pltpu.VMEM((CH, H), jnp.float32),
            pltpu.VMEM((CH, H), jnp.float32),
            pltpu.VMEM_SHARED((NT, H), jnp.float32),
            pltpu.VMEM_SHARED((NT, H), jnp.float32),
            pltpu.SemaphoreType.DMA,
        ],
    )
    return f(_sc_scatter_body)(wm, am, dstm, zn16)


def _node_body(x_ref, wl_ref, bl_ref, wr_ref, br_ref, xl_ref, xr_ref):
    x = x_ref[...]
    xl_ref[...] = jnp.dot(x, wl_ref[...],
                          preferred_element_type=jnp.float32) + bl_ref[...]
    xr_ref[...] = jnp.dot(x, wr_ref[...],
                          preferred_element_type=jnp.float32) + br_ref[...]


def _tc_node(x, wl, bl, wr, br):
    return pl.pallas_call(
        _node_body,
        out_shape=[jax.ShapeDtypeStruct((N, H), jnp.float32),
                   jax.ShapeDtypeStruct((N, H), jnp.float32)],
    )(x, wl, bl, wr, br)


_EB = 2048


def _edge_body(gs_ref, hd_ref, ea_ref, we_ref, att_ref, t16_ref, r16_ref,
               w_ref, a_ref):
    gs = gs_ref[...]
    m = gs + hd_ref[...] + ea_ref[...] * we_ref[...]
    m = jnp.where(m >= 0, m, 0.2 * m)
    alpha = jnp.dot(m * att_ref[...], t16_ref[...],
                    preferred_element_type=jnp.float32)
    a8 = jnp.exp(alpha)
    a128 = jnp.dot(a8, r16_ref[...],
                   preferred_element_type=jnp.float32)
    w_ref[...] = gs * a128
    a_ref[...] = a128


def _tc_edge(gsf, hdf, eaf, we128, att128, t16, r16):
    grid = (FR // _EB,)
    fb = pl.BlockSpec((_EB, 128), lambda i: (i, 0))
    return pl.pallas_call(
        _edge_body,
        grid=grid,
        in_specs=[
            fb, fb, fb,
            pl.BlockSpec((1, 128), lambda i: (0, 0)),
            pl.BlockSpec((1, 128), lambda i: (0, 0)),
            pl.BlockSpec((128, 8), lambda i: (0, 0)),
            pl.BlockSpec((8, 128), lambda i: (0, 0)),
        ],
        out_specs=[fb, fb],
        out_shape=[jax.ShapeDtypeStruct((FR, 128), jnp.float32),
                   jax.ShapeDtypeStruct((FR, 128), jnp.float32)],
    )(gsf, hdf, eaf, we128, att128, t16, r16)


_NB = 2000


def _comb1_body(m0, m1, d0, d1, dg0, dg1, es0, es1, xl, xr,
                we, att, b1, wl2, bl2, wr2, br2,
                xl2o, xr2o, lao):
    deg = dg0[...] + dg1[...]
    la = (es0[...] + es1[...]) / jnp.maximum(deg, 1.0)
    xlv = xl[...]
    ms = xlv + xr[...] + la * we[...]
    ms = jnp.where(ms >= 0, ms, 0.2 * ms)
    aself = jnp.exp(jnp.sum(ms * att[...], axis=1, keepdims=True))
    msg = m0[...] + m1[...] + aself * xlv
    den = d0[...] + d1[...] + aself
    h = msg / (den + 1e-16) + b1[...]
    h = jnp.where(h > 0, h, jnp.exp(h) - 1.0)
    xl2o[...] = jnp.dot(h, wl2[...],
                        preferred_element_type=jnp.float32) + bl2[...]
    xr2o[...] = jnp.dot(h, wr2[...],
                        preferred_element_type=jnp.float32) + br2[...]
    lao[...] = la


def _tc_comb1(m0, m1, d0, d1, dg0, dg1, es0, es1, xl, xr,
              we, att, b1, wl2, bl2, wr2, br2):
    grid = (N // _NB,)
    nb = lambda w: pl.BlockSpec((_NB, w), lambda i: (i, 0))
    full = lambda a, b: pl.BlockSpec((a, b), lambda i: (0, 0))
    return pl.pallas_call(
        _comb1_body,
        grid=grid,
        in_specs=[
            nb(H), nb(H), nb(1), nb(1), nb(1), nb(1), nb(1), nb(1),
            nb(H), nb(H),
            full(1, H), full(1, H), full(1, H),
            full(H, H), full(1, H), full(H, H), full(1, H),
        ],
        out_specs=[nb(H), nb(H), nb(1)],
        out_shape=[jax.ShapeDtypeStruct((N, H), jnp.float32),
                   jax.ShapeDtypeStruct((N, H), jnp.float32),
                   jax.ShapeDtypeStruct((N, 1), jnp.float32)],
    )(m0, m1, d0, d1, dg0, dg1, es0, es1, xl, xr,
      we, att, b1, wl2, bl2, wr2, br2)


def _final_body(m0, m1, d0, d1, la, xl, xr, bt, we, att, b2, w3, b3,
                out_ref):
    xlv = xl[...]
    ms = xlv + xr[...] + la[...] * we[...]
    ms = jnp.where(ms >= 0, ms, 0.2 * ms)
    aself = jnp.exp(jnp.sum(ms * att[...], axis=1, keepdims=True))
    msg = m0[...] + m1[...] + aself * xlv
    den = d0[...] + d1[...] + aself
    h = msg / (den + 1e-16) + b2[...]
    h = jnp.where(h > 0, h, jnp.exp(h) - 1.0)
    onehot = (bt[...] == lax.broadcasted_iota(jnp.int32, (G, N), 0)
              ).astype(jnp.float32)
    cnt = jnp.sum(onehot, axis=1, keepdims=True)
    pooled = jax.lax.dot_general(onehot, h, (((1,), (0,)), ((), ())),
                                 preferred_element_type=jnp.float32)
    pooled = pooled / jnp.maximum(cnt, 1.0)
    out_ref[...] = jnp.dot(pooled, w3[...],
                           preferred_element_type=jnp.float32) + b3[...]


def _tc_final(m0, m1, d0, d1, la, xl, xr, bt, we, att, b2, w3, b3):
    return pl.pallas_call(
        _final_body,
        out_shape=jax.ShapeDtypeStruct((G, 1), jnp.float32),
    )(m0, m1, d0, d1, la, xl, xr, bt, we, att, b2, w3, b3)


def kernel(x, edge_index, edge_attr, batch,
           Wl1, bl1, Wr1, br1, We1, att1, b1,
           Wl2, bl2, Wr2, br2, We2, att2, b2,
           W3, b3):
    f32 = jnp.float32
    src = edge_index[0]
    dst = edge_index[1]

    npad_e = EP - E
    padtgt = (N + (jnp.arange(npad_e, dtype=jnp.int32) % NPAD))
    src_p = jnp.concatenate([src, padtgt])
    dst_p = jnp.concatenate([dst, padtgt])
    ea_p = jnp.concatenate([edge_attr[:, 0], jnp.zeros((npad_e,), f32)])
    srcm = src_p.reshape(ROWS, S)
    dstm = dst_p.reshape(ROWS, S)
    eam = ea_p.reshape(ROWS, S)
    ea_flat = (ea_p[:, None] * jnp.ones((1, H), f32)).reshape(FR, 128)

    lane = jnp.arange(128, dtype=jnp.int32)
    t16 = (lane[:, None] // H == jnp.arange(8)[None, :]).astype(f32)
    r16 = t16.T

    zn = jnp.zeros((NT,), f32)
    zn16 = jnp.zeros((NT, H), f32)
    zrows = jnp.zeros((NPAD, H), f32)

    H2 = Wl2.shape[1]
    pw = ((0, 0), (0, H - H2))
    Wl2p = jnp.pad(Wl2, pw)
    Wr2p = jnp.pad(Wr2, pw)
    bl2p = jnp.pad(bl2, (0, H - H2)).reshape(1, H)
    br2p = jnp.pad(br2, (0, H - H2)).reshape(1, H)
    We2p = jnp.pad(We2, pw).reshape(1, H)
    att2p = jnp.pad(att2, (0, H - H2)).reshape(1, H)
    b2p = jnp.pad(b2, (0, H - H2)).reshape(1, H)
    W3p = jnp.pad(W3, ((0, H - H2), (0, 0)))

    We1r = We1.reshape(1, H)
    att1r = att1.reshape(1, H)
    b1r = b1.reshape(1, H)
    bl1r = bl1.reshape(1, H)
    br1r = br1.reshape(1, H)

    xl1, xr1 = _tc_node(x, Wl1, bl1r, Wr1, br1r)
    xl1t = jnp.concatenate([xl1, zrows])
    xr1t = jnp.concatenate([xr1, zrows])
    we1_128 = jnp.tile(We1r, (1, 8))
    att1_128 = jnp.tile(att1r, (1, 8))
    gs1, hd1, degp, easump = _sc_gather_deg(xl1t, xr1t, srcm, dstm, eam, zn)
    w1, a1 = _tc_edge(gs1.reshape(FR, 128), hd1.reshape(FR, 128), ea_flat,
                      we1_128, att1_128, t16, r16)
    msgp1, denp1 = _sc_scatter(w1.reshape(EP, H), a1.reshape(EP, H),
                               dstm, zn16)

    dg0 = degp[0, :N].reshape(N, 1)
    dg1 = degp[1, :N].reshape(N, 1)
    es0 = easump[0, :N].reshape(N, 1)
    es1 = easump[1, :N].reshape(N, 1)
    m0 = msgp1[0, :N]
    m1 = msgp1[1, :N]
    d0 = denp1[0, :N, 0].reshape(N, 1)
    d1 = denp1[1, :N, 0].reshape(N, 1)

    xl2, xr2, la = _tc_comb1(m0, m1, d0, d1, dg0, dg1, es0, es1, xl1, xr1,
                             We1r, att1r, b1r, Wl2p, bl2p, Wr2p, br2p)

    xl2t = jnp.concatenate([xl2, zrows])
    xr2t = jnp.concatenate([xr2, zrows])
    we2_128 = jnp.tile(We2p, (1, 8))
    att2_128 = jnp.tile(att2p, (1, 8))
    gs2, hd2 = _sc_gather(xl2t, xr2t, srcm, dstm)
    w2, a2 = _tc_edge(gs2.reshape(FR, 128), hd2.reshape(FR, 128), ea_flat,
                      we2_128, att2_128, t16, r16)
    msgp2, denp2 = _sc_scatter(w2.reshape(EP, H), a2.reshape(EP, H),
                               dstm, zn16)

    m0b = msgp2[0, :N]
    m1b = msgp2[1, :N]
    d0b = denp2[0, :N, 0].reshape(N, 1)
    d1b = denp2[1, :N, 0].reshape(N, 1)

    bt = batch.reshape(1, N)
    out = _tc_final(m0b, m1b, d0b, d1b, la, xl2, xr2, bt,
                    We2p, att2p, b2p, W3p, b3.reshape(1, 1))
    return out


# speedup vs baseline: 32.5107x; 1.1998x over previous
import functools

import jax
import jax.numpy as jnp
from jax import lax
from jax.experimental import pallas as pl
from jax.experimental.pallas import tpu as pltpu
from jax.experimental.pallas import tpu_sc as plsc

N = 10000
E = 320000
D = 128
H = 16
G = 64

NPAD = 16
NT = N + NPAD
NC = 2
NS = 16
NW = NC * NS
S = 128
KJ = 8
CH = KJ * S
EP = 327680
ROWS = EP // S
RPW = ROWS // NW
NCHUNK = RPW // KJ

FR = EP * H // 128

_mesh = plsc.VectorSubcoreMesh(core_axis_name="c", subcore_axis_name="s")
_sc_params = pltpu.CompilerParams(use_tc_tiling_on_sc=False)


def _sc_gather_body(xl_hbm, xr_hbm, srcm, dstm, gs_hbm, hd_hbm,
                    idxs0, idxs1, idxd0, idxd1, gsb0, gsb1, hdb0, hdb1,
                    sem1, sem2):
    c = lax.axis_index("c")
    s = lax.axis_index("s")
    wid = s * NC + c
    row0 = wid * RPW
    idxs = (idxs0, idxs1)
    idxd = (idxd0, idxd1)
    gsb = (gsb0, gsb1)
    hdb = (hdb0, hdb1)
    cps = {}

    def fire(ch):
        b = ch % 2
        r = row0 + ch * KJ
        pltpu.sync_copy(srcm.at[pl.ds(r, KJ)], idxs[b])
        pltpu.sync_copy(dstm.at[pl.ds(r, KJ)], idxd[b])
        lst = []
        for j in range(KJ):
            lst.append(pltpu.async_copy(
                xl_hbm.at[idxs[b].at[j]], gsb[b].at[pl.ds(j * S, S)], sem1))
            lst.append(pltpu.async_copy(
                xr_hbm.at[idxd[b].at[j]], hdb[b].at[pl.ds(j * S, S)], sem2))
        cps[ch] = lst

    def drain(ch):
        b = ch % 2
        e0 = (row0 + ch * KJ) * S
        for cp in cps.pop(ch):
            cp.wait()
        pltpu.sync_copy(gsb[b], gs_hbm.at[pl.ds(e0, CH)])
        pltpu.sync_copy(hdb[b], hd_hbm.at[pl.ds(e0, CH)])

    fire(0)
    for ch in range(NCHUNK):
        if ch + 1 < NCHUNK:
            fire(ch + 1)
        drain(ch)


def _sc_gather(xl_t, xr_t, srcm, dstm):
    f = functools.partial(
        pl.kernel,
        out_type=[
            jax.ShapeDtypeStruct((EP, H), jnp.float32),
            jax.ShapeDtypeStruct((EP, H), jnp.float32),
        ],
        mesh=_mesh,
        compiler_params=_sc_params,
        scratch_types=[
            pltpu.VMEM((KJ, S), jnp.int32),
            pltpu.VMEM((KJ, S), jnp.int32),
            pltpu.VMEM((KJ, S), jnp.int32),
            pltpu.VMEM((KJ, S), jnp.int32),
            pltpu.VMEM((CH, H), jnp.float32),
            pltpu.VMEM((CH, H), jnp.float32),
            pltpu.VMEM((CH, H), jnp.float32),
            pltpu.VMEM((CH, H), jnp.float32),
            pltpu.SemaphoreType.DMA,
            pltpu.SemaphoreType.DMA,
        ],
    )
    return f(_sc_gather_body)(xl_t, xr_t, srcm, dstm)


def _sc_scatter_body(wm, am, dstm, zn16, msgp, denp,
                     idxd0, idxd1, ab0, ab1, wb0, wb1, macc, dnacc, sem1):
    c = lax.axis_index("c")
    s = lax.axis_index("s")
    wid = s * NC + c
    row0 = wid * RPW
    idxd = (idxd0, idxd1)
    ab = (ab0, ab1)
    wb = (wb0, wb1)
    cps = {}

    @pl.when(s == 0)
    def _():
        pltpu.sync_copy(zn16, macc)
        pltpu.sync_copy(zn16, dnacc)
    plsc.subcore_barrier()

    def fire(ch):
        b = ch % 2
        r = row0 + ch * KJ
        e0 = r * S
        cps[ch] = [
            pltpu.async_copy(dstm.at[pl.ds(r, KJ)], idxd[b], sem1),
            pltpu.async_copy(am.at[pl.ds(e0, CH)], ab[b], sem1),
            pltpu.async_copy(wm.at[pl.ds(e0, CH)], wb[b], sem1),
        ]

    def drain(ch):
        b = ch % 2
        for cp in cps.pop(ch):
            cp.wait()
        for j in range(KJ):
            pltpu.sync_copy(wb[b].at[pl.ds(j * S, S)], macc.at[idxd[b].at[j]],
                            add=True)
            pltpu.sync_copy(ab[b].at[pl.ds(j * S, S)], dnacc.at[idxd[b].at[j]],
                            add=True)

    fire(0)
    for ch in range(NCHUNK):
        if ch + 1 < NCHUNK:
            fire(ch + 1)
        drain(ch)

    plsc.subcore_barrier()

    @pl.when(s == 0)
    def _():
        pltpu.sync_copy(macc, msgp.at[c])
        pltpu.sync_copy(dnacc, denp.at[c])


def _sc_scatter(wm, am, dstm, zn16):
    f = functools.partial(
        pl.kernel,
        out_type=[
            jax.ShapeDtypeStruct((NC, NT, H), jnp.float32),
            jax.ShapeDtypeStruct((NC, NT, H), jnp.float32),
        ],
        mesh=_mesh,
        compiler_params=_sc_params,
        scratch_types=[
            pltpu.VMEM((KJ, S), jnp.int32),
            pltpu.VMEM((KJ, S), jnp.int32),
            pltpu.VMEM((CH, H), jnp.float32),
            pltpu.VMEM((CH, H), jnp.float32),
            pltpu.VMEM((CH, H), jnp.float32),
            pltpu.VMEM((CH, H), jnp.float32),
            pltpu.VMEM_SHARED((NT, H), jnp.float32),
            pltpu.VMEM_SHARED((NT, H), jnp.float32),
            pltpu.SemaphoreType.DMA,
        ],
    )
    return f(_sc_scatter_body)(wm, am, dstm, zn16)


def _node_body(x_ref, wl_ref, bl_ref, wr_ref, br_ref, xl_ref, xr_ref):
    x = x_ref[...]
    xl_ref[...] = jnp.dot(x, wl_ref[...],
                          preferred_element_type=jnp.float32) + bl_ref[...]
    xr_ref[...] = jnp.dot(x, wr_ref[...],
                          preferred_element_type=jnp.float32) + br_ref[...]


def _tc_node(x, wl, bl, wr, br):
    return pl.pallas_call(
        _node_body,
        out_shape=[jax.ShapeDtypeStruct((N, H), jnp.float32),
                   jax.ShapeDtypeStruct((N, H), jnp.float32)],
    )(x, wl, bl, wr, br)


_EB = 2048


def _edge_body(gs_ref, hd_ref, eat_ref, att_ref, r16w_ref, t16_ref, r16_ref,
               r16a0_ref, r16e2_ref, w_ref, a_ref):
    gs = gs_ref[...]
    eat = eat_ref[...]
    eaw = lax.dot_general(eat, r16w_ref[...], (((0,), (0,)), ((), ())),
                          preferred_element_type=jnp.float32)
    m = gs + hd_ref[...] + eaw
    m = jnp.where(m >= 0, m, 0.2 * m)
    alpha = jnp.dot(m * att_ref[...], t16_ref[...],
                    preferred_element_type=jnp.float32)
    a8 = jnp.exp(alpha)
    a128 = jnp.dot(a8, r16_ref[...],
                   preferred_element_type=jnp.float32)
    w_ref[...] = gs * a128
    pay = jnp.dot(a8, r16a0_ref[...], preferred_element_type=jnp.float32)
    pay = pay + lax.dot_general(eat, r16e2_ref[...], (((0,), (0,)), ((), ())),
                                preferred_element_type=jnp.float32)
    a_ref[...] = pay + \
        jnp.where(lax.broadcasted_iota(jnp.int32, (_EB, 128), 1) % H == 1,
                  1.0, 0.0)


def _tc_edge(gsf, hdf, eat, att128, r16w, t16, r16, r16a0, r16e2):
    grid = (FR // _EB,)
    fb = pl.BlockSpec((_EB, 128), lambda i: (i, 0))
    return pl.pallas_call(
        _edge_body,
        grid=grid,
        in_specs=[
            fb, fb,
            pl.BlockSpec((8, _EB), lambda i: (0, i)),
            pl.BlockSpec((1, 128), lambda i: (0, 0)),
            pl.BlockSpec((8, 128), lambda i: (0, 0)),
            pl.BlockSpec((128, 8), lambda i: (0, 0)),
            pl.BlockSpec((8, 128), lambda i: (0, 0)),
            pl.BlockSpec((8, 128), lambda i: (0, 0)),
            pl.BlockSpec((8, 128), lambda i: (0, 0)),
        ],
        out_specs=[fb, fb],
        out_shape=[jax.ShapeDtypeStruct((FR, 128), jnp.float32),
                   jax.ShapeDtypeStruct((FR, 128), jnp.float32)],
    )(gsf, hdf, eat, att128, r16w, t16, r16, r16a0, r16e2)


_NB = 2000


def _comb1_body(m0, m1, d0, d1, dg0, dg1, es0, es1, xl, xr,
                we, att, b1, wl2, bl2, wr2, br2,
                xl2o, xr2o, lao):
    deg = dg0[...] + dg1[...]
    la = (es0[...] + es1[...]) / jnp.maximum(deg, 1.0)
    xlv = xl[...]
    ms = xlv + xr[...] + la * we[...]
    ms = jnp.where(ms >= 0, ms, 0.2 * ms)
    aself = jnp.exp(jnp.sum(ms * att[...], axis=1, keepdims=True))
    msg = m0[...] + m1[...] + aself * xlv
    den = d0[...] + d1[...] + aself
    h = msg / (den + 1e-16) + b1[...]
    h = jnp.where(h > 0, h, jnp.exp(h) - 1.0)
    xl2o[...] = jnp.dot(h, wl2[...],
                        preferred_element_type=jnp.float32) + bl2[...]
    xr2o[...] = jnp.dot(h, wr2[...],
                        preferred_element_type=jnp.float32) + br2[...]
    lao[...] = la


def _tc_comb1(m0, m1, d0, d1, dg0, dg1, es0, es1, xl, xr,
              we, att, b1, wl2, bl2, wr2, br2):
    grid = (N // _NB,)
    nb = lambda w: pl.BlockSpec((_NB, w), lambda i: (i, 0))
    full = lambda a, b: pl.BlockSpec((a, b), lambda i: (0, 0))
    return pl.pallas_call(
        _comb1_body,
        grid=grid,
        in_specs=[
            nb(H), nb(H), nb(1), nb(1), nb(1), nb(1), nb(1), nb(1),
            nb(H), nb(H),
            full(1, H), full(1, H), full(1, H),
            full(H, H), full(1, H), full(H, H), full(1, H),
        ],
        out_specs=[nb(H), nb(H), nb(1)],
        out_shape=[jax.ShapeDtypeStruct((N, H), jnp.float32),
                   jax.ShapeDtypeStruct((N, H), jnp.float32),
                   jax.ShapeDtypeStruct((N, 1), jnp.float32)],
    )(m0, m1, d0, d1, dg0, dg1, es0, es1, xl, xr,
      we, att, b1, wl2, bl2, wr2, br2)


def _final_body(m0, m1, d0, d1, la, xl, xr, bt, we, att, b2, w3, b3,
                out_ref):
    xlv = xl[...]
    ms = xlv + xr[...] + la[...] * we[...]
    ms = jnp.where(ms >= 0, ms, 0.2 * ms)
    aself = jnp.exp(jnp.sum(ms * att[...], axis=1, keepdims=True))
    msg = m0[...] + m1[...] + aself * xlv
    den = d0[...] + d1[...] + aself
    h = msg / (den + 1e-16) + b2[...]
    h = jnp.where(h > 0, h, jnp.exp(h) - 1.0)
    onehot = (bt[...] == lax.broadcasted_iota(jnp.int32, (G, N), 0)
              ).astype(jnp.float32)
    cnt = jnp.sum(onehot, axis=1, keepdims=True)
    pooled = jax.lax.dot_general(onehot, h, (((1,), (0,)), ((), ())),
                                 preferred_element_type=jnp.float32)
    pooled = pooled / jnp.maximum(cnt, 1.0)
    out_ref[...] = jnp.dot(pooled, w3[...],
                           preferred_element_type=jnp.float32) + b3[...]


def _tc_final(m0, m1, d0, d1, la, xl, xr, bt, we, att, b2, w3, b3):
    return pl.pallas_call(
        _final_body,
        out_shape=jax.ShapeDtypeStruct((G, 1), jnp.float32),
    )(m0, m1, d0, d1, la, xl, xr, bt, we, att, b2, w3, b3)


def kernel(x, edge_index, edge_attr, batch,
           Wl1, bl1, Wr1, br1, We1, att1, b1,
           Wl2, bl2, Wr2, br2, We2, att2, b2,
           W3, b3):
    f32 = jnp.float32
    src = edge_index[0]
    dst = edge_index[1]

    npad_e = EP - E
    padtgt = (N + (jnp.arange(npad_e, dtype=jnp.int32) % NPAD))
    src_p = jnp.concatenate([src, padtgt])
    dst_p = jnp.concatenate([dst, padtgt])
    ea_p = jnp.concatenate([edge_attr[:, 0], jnp.zeros((npad_e,), f32)])
    srcm = src_p.reshape(ROWS, S)
    dstm = dst_p.reshape(ROWS, S)
    eat = ea_p.reshape(FR, 8).T

    lane = jnp.arange(128, dtype=jnp.int32)
    grp = lane[:, None] // H == jnp.arange(8)[None, :]
    t16 = grp.astype(f32)
    r16 = t16.T
    r16a0 = r16 * (lane[None, :] % H == 0)
    r16e2 = r16 * (lane[None, :] % H == 2)

    zn16 = jnp.zeros((NT, H), f32)
    zrows = jnp.zeros((NPAD, H), f32)

    H2 = Wl2.shape[1]
    pw = ((0, 0), (0, H - H2))
    Wl2p = jnp.pad(Wl2, pw)
    Wr2p = jnp.pad(Wr2, pw)
    bl2p = jnp.pad(bl2, (0, H - H2)).reshape(1, H)
    br2p = jnp.pad(br2, (0, H - H2)).reshape(1, H)
    We2p = jnp.pad(We2, pw).reshape(1, H)
    att2p = jnp.pad(att2, (0, H - H2)).reshape(1, H)
    b2p = jnp.pad(b2, (0, H - H2)).reshape(1, H)
    W3p = jnp.pad(W3, ((0, H - H2), (0, 0)))

    We1r = We1.reshape(1, H)
    att1r = att1.reshape(1, H)
    b1r = b1.reshape(1, H)
    bl1r = bl1.reshape(1, H)
    br1r = br1.reshape(1, H)

    att1_128 = jnp.tile(att1r, (1, 8))
    att2_128 = jnp.tile(att2p, (1, 8))
    r16w1 = r16 * jnp.tile(We1r, (1, 8))
    r16w2 = r16 * jnp.tile(We2p, (1, 8))

    xl1, xr1 = _tc_node(x, Wl1, bl1r, Wr1, br1r)
    xl1t = jnp.concatenate([xl1, zrows])
    xr1t = jnp.concatenate([xr1, zrows])
    gs1, hd1 = _sc_gather(xl1t, xr1t, srcm, dstm)
    w1, a1 = _tc_edge(gs1.reshape(FR, 128), hd1.reshape(FR, 128), eat,
                      att1_128, r16w1, t16, r16, r16a0, r16e2)
    msgp1, denp1 = _sc_scatter(w1.reshape(EP, H), a1.reshape(EP, H),
                               dstm, zn16)

    dg0 = denp1[0, :N, 1].reshape(N, 1)
    dg1 = denp1[1, :N, 1].reshape(N, 1)
    es0 = denp1[0, :N, 2].reshape(N, 1)
    es1 = denp1[1, :N, 2].reshape(N, 1)
    m0 = msgp1[0, :N]
    m1 = msgp1[1, :N]
    d0 = denp1[0, :N, 0].reshape(N, 1)
    d1 = denp1[1, :N, 0].reshape(N, 1)

    xl2, xr2, la = _tc_comb1(m0, m1, d0, d1, dg0, dg1, es0, es1, xl1, xr1,
                             We1r, att1r, b1r, Wl2p, bl2p, Wr2p, br2p)

    xl2t = jnp.concatenate([xl2, zrows])
    xr2t = jnp.concatenate([xr2, zrows])
    gs2, hd2 = _sc_gather(xl2t, xr2t, srcm, dstm)
    w2, a2 = _tc_edge(gs2.reshape(FR, 128), hd2.reshape(FR, 128), eat,
                      att2_128, r16w2, t16, r16, r16a0, r16e2)
    msgp2, denp2 = _sc_scatter(w2.reshape(EP, H), a2.reshape(EP, H),
                               dstm, zn16)

    m0b = msgp2[0, :N]
    m1b = msgp2[1, :N]
    d0b = denp2[0, :N, 0].reshape(N, 1)
    d1b = denp2[1, :N, 0].reshape(N, 1)

    bt = batch.reshape(1, N)
    out = _tc_final(m0b, m1b, d0b, d1b, la, xl2, xr2, bt,
                    We2p, att2p, b2p, W3p, b3.reshape(1, 1))
    return out
